# baseline (speedup 1.0000x reference)
import functools

import jax
import jax.numpy as jnp
from jax import lax
from jax.experimental import pallas as pl
from jax.experimental.pallas import tpu as pltpu
from jax.experimental.pallas import tpu_sc as plsc

NC, NS, L = 2, 16, 16
NW = NC * NS
HIDDEN = 128
HCHUNKS = HIDDEN // L


def _rsqrt(x):
    xi = plsc.bitcast(x, jnp.int32)
    yi = jnp.int32(0x5F3759DF) - (xi >> 1)
    y = plsc.bitcast(yi, jnp.float32)
    for _ in range(3):
        y = y * (1.5 - 0.5 * x * y * y)
    return y


def _make_sc_kernel(n_tokens, seq_len):
    b_per_w = n_tokens // NW
    mesh = plsc.VectorSubcoreMesh(
        core_axis_name="c", subcore_axis_name="s", num_cores=NC, num_subcores=NS
    )

    @functools.partial(
        pl.kernel,
        mesh=mesh,
        compiler_params=pltpu.CompilerParams(needs_layout_passes=False),
        out_type=jax.ShapeDtypeStruct((n_tokens, HIDDEN), jnp.float32),
        scratch_types=[
            pltpu.VMEM((b_per_w,), jnp.int32),
            pltpu.VMEM((b_per_w,), jnp.int32),
            pltpu.VMEM((b_per_w, HIDDEN), jnp.float32),
            pltpu.VMEM((b_per_w, HIDDEN), jnp.float32),
            pltpu.VMEM((b_per_w, HIDDEN), jnp.float32),
            pltpu.VMEM((HIDDEN,), jnp.float32),
            pltpu.VMEM((HIDDEN,), jnp.float32),
            pltpu.SemaphoreType.DMA,
            pltpu.SemaphoreType.DMA,
        ],
    )
    def sc_kernel(ids_hbm, tt_hbm, token_hbm, pos_hbm, type_hbm, g_hbm, b_hbm,
                  out_hbm, idx_v, tt_v, rows_v, pos_v, typ_v, g_v, b_v,
                  sem0, sem1):
        wid = lax.axis_index("s") * NC + lax.axis_index("c")
        base = wid * b_per_w

        pltpu.sync_copy(ids_hbm.at[pl.ds(base, b_per_w)], idx_v)
        pltpu.sync_copy(tt_hbm.at[pl.ds(base, b_per_w)], tt_v)

        copies = []
        for j in range(b_per_w // 128):
            sem = sem0 if j % 2 == 0 else sem1
            copies.append(pltpu.async_copy(
                token_hbm.at[idx_v.at[pl.ds(j * 128, 128)]],
                rows_v.at[pl.ds(j * 128, 128)], sem))
            copies.append(pltpu.async_copy(
                type_hbm.at[tt_v.at[pl.ds(j * 128, 128)]],
                typ_v.at[pl.ds(j * 128, 128)], sem))

        pos_base = lax.rem(base, seq_len)
        pltpu.sync_copy(pos_hbm.at[pl.ds(pos_base, b_per_w)], pos_v)
        pltpu.sync_copy(g_hbm, g_v)
        pltpu.sync_copy(b_hbm, b_v)
        for c in copies:
            c.wait()

        gs = [g_v[pl.ds(h * L, L)] for h in range(HCHUNKS)]
        bs = [b_v[pl.ds(h * L, L)] for h in range(HCHUNKS)]
        inv_h = jnp.float32(1.0 / HIDDEN)

        def body(i, _):
            vals = []
            s = jnp.zeros((L,), jnp.float32)
            sq = jnp.zeros((L,), jnp.float32)
            for h in range(HCHUNKS):
                v = (rows_v[i, pl.ds(h * L, L)]
                     + pos_v[i, pl.ds(h * L, L)]
                     + typ_v[i, pl.ds(h * L, L)])
                vals.append(v)
                s = s + v
                sq = sq + v * v
            mean = jnp.sum(s) * inv_h
            ex2 = jnp.sum(sq) * inv_h
            var = ex2 - mean * mean
            inv = _rsqrt(jnp.full((L,), var + 1e-12, jnp.float32))
            mean_v = jnp.full((L,), mean, jnp.float32)
            for h in range(HCHUNKS):
                rows_v[i, pl.ds(h * L, L)] = (vals[h] - mean_v) * inv * gs[h] + bs[h]
            return 0

        lax.fori_loop(0, b_per_w, body, 0)

        pltpu.sync_copy(rows_v, out_hbm.at[pl.ds(base, b_per_w)])

    return sc_kernel


def kernel(input_ids, token_type_ids, token_table, pos_table, type_table,
           ln_gamma, ln_beta):
    b, s = input_ids.shape
    n = b * s
    ids = input_ids.reshape(n).astype(jnp.int32)
    tt = token_type_ids.reshape(n).astype(jnp.int32)
    sc = _make_sc_kernel(n, s)
    out = sc(ids, tt, token_table, pos_table, type_table, ln_gamma, ln_beta)
    return out.reshape(b, s, HIDDEN)
